# Initial kernel scaffold; baseline (speedup 1.0000x reference)
import jax
import jax.numpy as jnp
from jax.experimental import pallas as pl


def kernel(t_list, t_hscore, W1, b1, W2, b2):
    raise NotImplementedError("write your pallas kernel here")


# speedup vs baseline: 1.3727x; 1.3727x over previous
import functools

import jax
import jax.numpy as jnp
from jax import lax
from jax.experimental import pallas as pl
from jax.experimental.pallas import tpu as pltpu
from jax.experimental.pallas import tpu_sc as plsc

L = 16
NC, NS = 2, 16
NW = NC * NS
B = 16384
BPW = B // NW


def _body(table_hbm, idx_hbm, params_hbm, out_hbm, idx_v, vals_v, out_v,
          params_v, sem):
    wid = lax.axis_index("s") * NC + lax.axis_index("c")
    base = wid * BPW

    pltpu.sync_copy(params_hbm, params_v)
    pltpu.sync_copy(idx_hbm.at[pl.ds(base, BPW)], idx_v)

    pltpu.async_copy(table_hbm.at[idx_v], vals_v, sem).wait()

    w1 = params_v[pl.ds(0, L)]
    b1 = params_v[pl.ds(L, L)]
    w2 = params_v[pl.ds(2 * L, L)]
    b2 = params_v[pl.ds(3 * L, L)]

    for j in range(BPW // L):
        x = vals_v[pl.ds(j * L, L)]
        out_v[pl.ds(j * L, L)] = w2 * jnp.maximum(w1 * x + b1, 0.0) + b2

    pltpu.sync_copy(out_v, out_hbm.at[pl.ds(base, BPW)])


@jax.jit
def _run(table, idx, params):
    mesh = plsc.VectorSubcoreMesh(core_axis_name="c", subcore_axis_name="s")
    return pl.kernel(
        _body,
        out_type=jax.ShapeDtypeStruct((B,), jnp.float32),
        mesh=mesh,
        scratch_types=[
            pltpu.VMEM((BPW,), jnp.int32),
            pltpu.VMEM((BPW,), jnp.float32),
            pltpu.VMEM((BPW,), jnp.float32),
            pltpu.VMEM((4 * L,), jnp.float32),
            pltpu.SemaphoreType.DMA,
        ],
    )(table, idx, params)


def kernel(t_list, t_hscore, W1, b1, W2, b2):
    params = jnp.concatenate([
        jnp.broadcast_to(W1.reshape(()), (L,)),
        jnp.broadcast_to(b1.reshape(()), (L,)),
        jnp.broadcast_to(W2.reshape(()), (L,)),
        jnp.broadcast_to(b2.reshape(()), (L,)),
    ]).astype(jnp.float32)
    idx = t_list.astype(jnp.int32)
    return _run(t_hscore.astype(jnp.float32), idx, params)


# speedup vs baseline: 5.8069x; 4.2304x over previous
import functools

import jax
import jax.numpy as jnp
from jax import lax
from jax.experimental import pallas as pl
from jax.experimental.pallas import tpu as pltpu
from jax.experimental.pallas import tpu_sc as plsc

L = 16
NC, NS = 2, 16
NW = NC * NS
B = 16384
BPW = B // NW
V = 100
VPAD = 112
NT = VPAD // L


def _body(table_hbm, idx_hbm, params_hbm, out_hbm, table_v, idx_v, out_v,
          params_v):
    wid = lax.axis_index("s") * NC + lax.axis_index("c")
    base = wid * BPW

    pltpu.sync_copy(table_hbm, table_v)
    pltpu.sync_copy(params_hbm, params_v)
    pltpu.sync_copy(idx_hbm.at[pl.ds(base, BPW)], idx_v)

    w1 = params_v[pl.ds(0, L)]
    b1 = params_v[pl.ds(L, L)]
    w2 = params_v[pl.ds(2 * L, L)]
    b2 = params_v[pl.ds(3 * L, L)]

    ftab = []
    for t in range(NT):
        x = table_v[pl.ds(t * L, L)]
        ftab.append(w2 * jnp.maximum(w1 * x + b1, 0.0) + b2)

    for j in range(BPW // L):
        idx = idx_v[pl.ds(j * L, L)]
        lane = lax.bitwise_and(idx, L - 1)
        grp = lax.shift_right_logical(idx, 4)
        acc = ftab[0].at[lane].get(mode="promise_in_bounds")
        for t in range(1, NT):
            g = ftab[t].at[lane].get(mode="promise_in_bounds")
            acc = jnp.where(grp == t, g, acc)
        out_v[pl.ds(j * L, L)] = acc

    pltpu.sync_copy(out_v, out_hbm.at[pl.ds(base, BPW)])


@jax.jit
def _run(table_pad, idx, params):
    mesh = plsc.VectorSubcoreMesh(core_axis_name="c", subcore_axis_name="s")
    return pl.kernel(
        _body,
        out_type=jax.ShapeDtypeStruct((B,), jnp.float32),
        mesh=mesh,
        scratch_types=[
            pltpu.VMEM((VPAD,), jnp.float32),
            pltpu.VMEM((BPW,), jnp.int32),
            pltpu.VMEM((BPW,), jnp.float32),
            pltpu.VMEM((4 * L,), jnp.float32),
        ],
    )(table_pad, idx, params)


def kernel(t_list, t_hscore, W1, b1, W2, b2):
    table_pad = jnp.pad(t_hscore.astype(jnp.float32), (0, VPAD - V))
    params = jnp.concatenate([
        jnp.broadcast_to(W1.reshape(()), (L,)),
        jnp.broadcast_to(b1.reshape(()), (L,)),
        jnp.broadcast_to(W2.reshape(()), (L,)),
        jnp.broadcast_to(b2.reshape(()), (L,)),
    ]).astype(jnp.float32)
    idx = t_list.astype(jnp.int32)
    return _run(table_pad, idx, params)


# speedup vs baseline: 5.8804x; 1.0127x over previous
import functools

import jax
import jax.numpy as jnp
from jax import lax
from jax.experimental import pallas as pl
from jax.experimental.pallas import tpu as pltpu
from jax.experimental.pallas import tpu_sc as plsc

L = 16
NC, NS = 2, 16
NW = NC * NS
B = 16384
BPW = B // NW
V = 100
VPAD = 112
NT = VPAD // L


def _body(table_hbm, idx_hbm, w1_hbm, b1_hbm, w2_hbm, b2_hbm, out_hbm,
          table_v, idx_v, out_v, w1_s, b1_s, w2_s, b2_s):
    wid = lax.axis_index("s") * NC + lax.axis_index("c")
    base = wid * BPW

    pltpu.sync_copy(table_hbm, table_v.at[pl.ds(0, V)])
    pltpu.sync_copy(w1_hbm, w1_s.at[pl.ds(0, 1)])
    pltpu.sync_copy(b1_hbm, b1_s.at[pl.ds(0, 1)])
    pltpu.sync_copy(w2_hbm, w2_s.at[pl.ds(0, 1)])
    pltpu.sync_copy(b2_hbm, b2_s.at[pl.ds(0, 1)])
    pltpu.sync_copy(idx_hbm.at[pl.ds(base, BPW)], idx_v)

    w1 = jnp.full((L,), w1_s[pl.ds(0, L)][0], jnp.float32)
    b1 = jnp.full((L,), b1_s[pl.ds(0, L)][0], jnp.float32)
    w2 = jnp.full((L,), w2_s[pl.ds(0, L)][0], jnp.float32)
    b2 = jnp.full((L,), b2_s[pl.ds(0, L)][0], jnp.float32)

    ftab = []
    for t in range(NT):
        x = table_v[pl.ds(t * L, L)]
        ftab.append(w2 * jnp.maximum(w1 * x + b1, 0.0) + b2)

    for j in range(BPW // L):
        idx = idx_v[pl.ds(j * L, L)]
        lane = lax.bitwise_and(idx, L - 1)
        grp = lax.shift_right_logical(idx, 4)
        acc = ftab[0].at[lane].get(mode="promise_in_bounds")
        for t in range(1, NT):
            g = ftab[t].at[lane].get(mode="promise_in_bounds")
            acc = jnp.where(grp == t, g, acc)
        out_v[pl.ds(j * L, L)] = acc

    pltpu.sync_copy(out_v, out_hbm.at[pl.ds(base, BPW)])


@jax.jit
def _run(table, idx, w1, b1, w2, b2):
    mesh = plsc.VectorSubcoreMesh(core_axis_name="c", subcore_axis_name="s")
    return pl.kernel(
        _body,
        out_type=jax.ShapeDtypeStruct((B,), jnp.float32),
        mesh=mesh,
        scratch_types=[
            pltpu.VMEM((VPAD,), jnp.float32),
            pltpu.VMEM((BPW,), jnp.int32),
            pltpu.VMEM((BPW,), jnp.float32),
            pltpu.VMEM((L,), jnp.float32),
            pltpu.VMEM((L,), jnp.float32),
            pltpu.VMEM((L,), jnp.float32),
            pltpu.VMEM((L,), jnp.float32),
        ],
    )(table, idx, w1, b1, w2, b2)


def kernel(t_list, t_hscore, W1, b1, W2, b2):
    return _run(t_hscore, t_list, W1.reshape((1,)), b1, W2.reshape((1,)), b2)


# speedup vs baseline: 6.0435x; 1.0277x over previous
import functools

import jax
import jax.numpy as jnp
from jax import lax
from jax.experimental import pallas as pl
from jax.experimental.pallas import tpu as pltpu
from jax.experimental.pallas import tpu_sc as plsc

L = 16
NC, NS = 2, 16
NW = NC * NS
B = 16384
BPW = B // NW
V = 100
VPAD = 112
NT = VPAD // L
SG = VPAD + L


def _body(stage_hbm, idx_hbm, out_hbm, stage_v, idx_v, out_v, sem1, sem2):
    wid = lax.axis_index("s") * NC + lax.axis_index("c")
    base = wid * BPW

    cp_idx = pltpu.async_copy(idx_hbm.at[pl.ds(base, BPW)], idx_v, sem1)
    cp_stage = pltpu.async_copy(stage_hbm, stage_v, sem2)
    cp_stage.wait()

    wv = stage_v[pl.ds(VPAD, L)]
    w1 = jnp.full((L,), wv[0], jnp.float32)
    b1 = jnp.full((L,), wv[1], jnp.float32)
    w2 = jnp.full((L,), wv[2], jnp.float32)
    b2 = jnp.full((L,), wv[3], jnp.float32)

    ftab = []
    for t in range(NT):
        x = stage_v[pl.ds(t * L, L)]
        ftab.append(w2 * jnp.maximum(w1 * x + b1, 0.0) + b2)

    cp_idx.wait()

    for j in range(BPW // L):
        idx = idx_v[pl.ds(j * L, L)]
        lane = lax.bitwise_and(idx, L - 1)
        grp = lax.shift_right_logical(idx, 4)
        acc = ftab[0].at[lane].get(mode="promise_in_bounds")
        for t in range(1, NT):
            g = ftab[t].at[lane].get(mode="promise_in_bounds")
            acc = jnp.where(grp == t, g, acc)
        out_v[pl.ds(j * L, L)] = acc

    pltpu.sync_copy(out_v, out_hbm.at[pl.ds(base, BPW)])


@jax.jit
def _run(stage, idx):
    mesh = plsc.VectorSubcoreMesh(core_axis_name="c", subcore_axis_name="s")
    return pl.kernel(
        _body,
        out_type=jax.ShapeDtypeStruct((B,), jnp.float32),
        mesh=mesh,
        scratch_types=[
            pltpu.VMEM((SG,), jnp.float32),
            pltpu.VMEM((BPW,), jnp.int32),
            pltpu.VMEM((BPW,), jnp.float32),
            pltpu.SemaphoreType.DMA,
            pltpu.SemaphoreType.DMA,
        ],
    )(stage, idx)


def kernel(t_list, t_hscore, W1, b1, W2, b2):
    stage = jnp.concatenate([
        t_hscore.astype(jnp.float32),
        jnp.zeros((VPAD - V,), jnp.float32),
        W1.reshape((1,)).astype(jnp.float32),
        b1.reshape((1,)).astype(jnp.float32),
        W2.reshape((1,)).astype(jnp.float32),
        b2.reshape((1,)).astype(jnp.float32),
        jnp.zeros((L - 4,), jnp.float32),
    ])
    return _run(stage, t_list.astype(jnp.int32))


# speedup vs baseline: 6.1119x; 1.0113x over previous
import functools

import jax
import jax.numpy as jnp
from jax import lax
from jax.experimental import pallas as pl
from jax.experimental.pallas import tpu as pltpu
from jax.experimental.pallas import tpu_sc as plsc

L = 16
NC, NS = 2, 16
NW = NC * NS
B = 16384
BPW = B // NW
V = 100
VPAD = 112
NT = VPAD // L
SG = VPAD + L


def _body(stage_hbm, idx_hbm, out_hbm, stage_v, idx_v, out_v, sem1, sem2):
    wid = lax.axis_index("s") * NC + lax.axis_index("c")
    base = wid * BPW

    cp_idx = pltpu.async_copy(idx_hbm.at[pl.ds(base, BPW)], idx_v, sem1)
    cp_stage = pltpu.async_copy(stage_hbm, stage_v, sem2)
    cp_stage.wait()

    wv = stage_v[pl.ds(VPAD, L)]
    w1 = jnp.full((L,), wv[0], jnp.float32)
    b1 = jnp.full((L,), wv[1], jnp.float32)
    w2 = jnp.full((L,), wv[2], jnp.float32)
    b2 = jnp.full((L,), wv[3], jnp.float32)

    ftab = []
    for t in range(NT):
        x = stage_v[pl.ds(t * L, L)]
        ftab.append(w2 * jnp.maximum(w1 * x + b1, 0.0) + b2)

    cp_idx.wait()

    UNROLL = 4

    def step(i, carry):
        for u in range(UNROLL):
            off = i * (UNROLL * L) + u * L
            idx = idx_v[pl.ds(off, L)]
            lane = lax.bitwise_and(idx, L - 1)
            grp = lax.shift_right_logical(idx, 4)
            acc = ftab[0].at[lane].get(mode="promise_in_bounds")
            for t in range(1, NT):
                g = ftab[t].at[lane].get(mode="promise_in_bounds")
                acc = jnp.where(grp == t, g, acc)
            out_v[pl.ds(off, L)] = acc
        return carry

    lax.fori_loop(0, BPW // (UNROLL * L), step, 0, unroll=False)

    pltpu.sync_copy(out_v, out_hbm.at[pl.ds(base, BPW)])


@jax.jit
def _run(stage, idx):
    mesh = plsc.VectorSubcoreMesh(core_axis_name="c", subcore_axis_name="s")
    return pl.kernel(
        _body,
        out_type=jax.ShapeDtypeStruct((B,), jnp.float32),
        mesh=mesh,
        scratch_types=[
            pltpu.VMEM((SG,), jnp.float32),
            pltpu.VMEM((BPW,), jnp.int32),
            pltpu.VMEM((BPW,), jnp.float32),
            pltpu.SemaphoreType.DMA,
            pltpu.SemaphoreType.DMA,
        ],
    )(stage, idx)


def kernel(t_list, t_hscore, W1, b1, W2, b2):
    stage = jnp.concatenate([
        t_hscore.astype(jnp.float32),
        jnp.zeros((VPAD - V,), jnp.float32),
        W1.reshape((1,)).astype(jnp.float32),
        b1.reshape((1,)).astype(jnp.float32),
        W2.reshape((1,)).astype(jnp.float32),
        b2.reshape((1,)).astype(jnp.float32),
        jnp.zeros((L - 4,), jnp.float32),
    ])
    return _run(stage, t_list.astype(jnp.int32))


# speedup vs baseline: 6.6878x; 1.0942x over previous
import functools

import jax
import jax.numpy as jnp
from jax import lax
from jax.experimental import pallas as pl
from jax.experimental.pallas import tpu as pltpu
from jax.experimental.pallas import tpu_sc as plsc

L = 16
NC, NS = 1, 16
NW = NC * NS
B = 16384
BPW = B // NW
V = 100
VPAD = 112
NT = VPAD // L
SG = VPAD + L


def _body(stage_hbm, idx_hbm, out_hbm, stage_v, idx_v, out_v, sem1, sem2):
    wid = lax.axis_index("s") * NC + lax.axis_index("c")
    base = wid * BPW

    cp_idx = pltpu.async_copy(idx_hbm.at[pl.ds(base, BPW)], idx_v, sem1)
    cp_stage = pltpu.async_copy(stage_hbm, stage_v, sem2)
    cp_stage.wait()

    wv = stage_v[pl.ds(VPAD, L)]
    w1 = jnp.full((L,), wv[0], jnp.float32)
    b1 = jnp.full((L,), wv[1], jnp.float32)
    w2 = jnp.full((L,), wv[2], jnp.float32)
    b2 = jnp.full((L,), wv[3], jnp.float32)

    ftab = []
    for t in range(NT):
        x = stage_v[pl.ds(t * L, L)]
        ftab.append(w2 * jnp.maximum(w1 * x + b1, 0.0) + b2)

    cp_idx.wait()

    UNROLL = 4

    def step(i, carry):
        for u in range(UNROLL):
            off = i * (UNROLL * L) + u * L
            idx = idx_v[pl.ds(off, L)]
            lane = lax.bitwise_and(idx, L - 1)
            grp = lax.shift_right_logical(idx, 4)
            acc = ftab[0].at[lane].get(mode="promise_in_bounds")
            for t in range(1, NT):
                g = ftab[t].at[lane].get(mode="promise_in_bounds")
                acc = jnp.where(grp == t, g, acc)
            out_v[pl.ds(off, L)] = acc
        return carry

    lax.fori_loop(0, BPW // (UNROLL * L), step, 0, unroll=False)

    pltpu.sync_copy(out_v, out_hbm.at[pl.ds(base, BPW)])


@jax.jit
def _run(stage, idx):
    mesh = plsc.VectorSubcoreMesh(core_axis_name="c", subcore_axis_name="s",
                                  num_cores=NC)
    return pl.kernel(
        _body,
        out_type=jax.ShapeDtypeStruct((B,), jnp.float32),
        mesh=mesh,
        scratch_types=[
            pltpu.VMEM((SG,), jnp.float32),
            pltpu.VMEM((BPW,), jnp.int32),
            pltpu.VMEM((BPW,), jnp.float32),
            pltpu.SemaphoreType.DMA,
            pltpu.SemaphoreType.DMA,
        ],
    )(stage, idx)


def kernel(t_list, t_hscore, W1, b1, W2, b2):
    stage = jnp.concatenate([
        t_hscore.astype(jnp.float32),
        jnp.zeros((VPAD - V,), jnp.float32),
        W1.reshape((1,)).astype(jnp.float32),
        b1.reshape((1,)).astype(jnp.float32),
        W2.reshape((1,)).astype(jnp.float32),
        b2.reshape((1,)).astype(jnp.float32),
        jnp.zeros((L - 4,), jnp.float32),
    ])
    return _run(stage, t_list.astype(jnp.int32))
